# Initial kernel scaffold; baseline (speedup 1.0000x reference)
import jax
import jax.numpy as jnp
from jax.experimental import pallas as pl


def kernel(complex_x, binder1_x, binder2_x, W, complex_mut_seqs, binder1_mut_seqs, binder2_mut_seqs, complex_wt_seq, binder1_wt_seq, binder2_wt_seq):
    raise NotImplementedError("write your pallas kernel here")


# speedup vs baseline: 3.3300x; 3.3300x over previous
import functools

import jax
import jax.numpy as jnp
from jax import lax
from jax.experimental import pallas as pl
from jax.experimental.pallas import tpu as pltpu
from jax.experimental.pallas import tpu_sc as plsc

LC, L1, L2, D, B, V = 1024, 512, 512, 128, 256, 21
LTOT = LC + L1 + L2
NW = 32
HALF = LTOT // 2
GRP = B // 16


def _tc_logprob_body(x_ref, w_ref, wt_ref, out_ref):
    logits = jnp.dot(x_ref[...], w_ref[...], preferred_element_type=jnp.float32)
    m = jnp.max(logits, axis=-1, keepdims=True)
    lse = jnp.log(jnp.sum(jnp.exp(logits - m), axis=-1, keepdims=True)) + m
    lp = logits - lse
    row = lax.broadcasted_iota(jnp.int32, (LTOT, 1), 0)
    sign = jnp.where(row < LC, -1.0, 1.0)
    lps = lp * sign
    col = lax.broadcasted_iota(jnp.int32, (LTOT, V), 1)
    wt_val = jnp.sum(jnp.where(col == wt_ref[...], lps, 0.0), axis=1,
                     keepdims=True)
    out_ref[...] = lps - wt_val


@functools.cache
def _make_sc_gather():
    mesh = plsc.VectorSubcoreMesh(core_axis_name="c", subcore_axis_name="s")

    @functools.partial(
        pl.kernel,
        mesh=mesh,
        out_type=jax.ShapeDtypeStruct((2, B), jnp.float32),
        compiler_params=pltpu.CompilerParams(needs_layout_passes=False),
        scratch_types=[
            pltpu.VMEM((HALF * 16,), jnp.int32),
            pltpu.VMEM((HALF * V,), jnp.float32),
            pltpu.VMEM((16,), jnp.float32),
        ],
    )
    def sc_gather(lp_hbm, seq_hbm, out_hbm, seq_v, lp_v, out_v):
        wid = lax.axis_index("s") * 2 + lax.axis_index("c")
        h = wid // 16
        g = wid % 16
        pltpu.sync_copy(seq_hbm.at[wid], seq_v)
        pltpu.sync_copy(lp_hbm.at[pl.ds(h * HALF * V, HALF * V)], lp_v)

        def body(l, acc):
            s = seq_v[pl.ds(l * 16, 16)]
            idx = s + jnp.full((16,), l * V, jnp.int32)
            return acc + plsc.load_gather(lp_v, [idx])

        acc = lax.fori_loop(0, HALF, body, jnp.zeros((16,), jnp.float32))
        out_v[...] = acc
        pltpu.sync_copy(out_v, out_hbm.at[h, pl.ds(g * 16, 16)])

    return sc_gather


def kernel(complex_x, binder1_x, binder2_x, W,
           complex_mut_seqs, binder1_mut_seqs, binder2_mut_seqs,
           complex_wt_seq, binder1_wt_seq, binder2_wt_seq):
    x_all = jnp.concatenate([complex_x, binder1_x, binder2_x], axis=0)
    wt_all = jnp.concatenate(
        [complex_wt_seq, binder1_wt_seq, binder2_wt_seq]
    ).astype(jnp.int32).reshape(LTOT, 1)

    lp_adj = pl.pallas_call(
        _tc_logprob_body,
        out_shape=jax.ShapeDtypeStruct((LTOT, V), jnp.float32),
    )(x_all, W, wt_all)

    seqs_all = jnp.concatenate(
        [complex_mut_seqs, binder1_mut_seqs, binder2_mut_seqs], axis=1
    ).astype(jnp.int32)
    seq_arr = (seqs_all.T.reshape(2, HALF, GRP, 16)
               .transpose(0, 2, 1, 3).reshape(NW, HALF * 16))

    partials = _make_sc_gather()(lp_adj.reshape(LTOT * V), seq_arr)
    return partials[0] + partials[1]


# speedup vs baseline: 5.6612x; 1.7001x over previous
import functools

import jax
import jax.numpy as jnp
from jax import lax
from jax.experimental import pallas as pl
from jax.experimental.pallas import tpu as pltpu
from jax.experimental.pallas import tpu_sc as plsc

LC, L1, L2, D, B, V = 1024, 512, 512, 128, 256, 21
LTOT = LC + L1 + L2
TS = V
QL = LTOT // 4
GS = 32


def _tc_logprob_body(cx_ref, b1_ref, b2_ref, w_ref, wt_ref, out_ref):
    x_all = jnp.concatenate([cx_ref[...], b1_ref[...], b2_ref[...]], axis=0)
    logits = jnp.dot(x_all, w_ref[...], preferred_element_type=jnp.float32)
    row = lax.broadcasted_iota(jnp.int32, (LTOT, 1), 0)
    sign = jnp.where(row < LC, -1.0, 1.0)
    col = lax.broadcasted_iota(jnp.int32, (LTOT, V), 1)
    wt_val = jnp.sum(jnp.where(col == wt_ref[...], logits, 0.0), axis=1,
                     keepdims=True)
    out_ref[...] = (logits - wt_val) * sign


@functools.cache
def _make_sc_gather():
    mesh = plsc.VectorSubcoreMesh(core_axis_name="c", subcore_axis_name="s")

    @functools.partial(
        pl.kernel,
        mesh=mesh,
        out_type=jax.ShapeDtypeStruct((4, B), jnp.float32),
        compiler_params=pltpu.CompilerParams(needs_layout_passes=False),
        scratch_types=[
            pltpu.VMEM((GS * QL,), jnp.int32),
            pltpu.VMEM((QL * TS,), jnp.float32),
            pltpu.VMEM((GS,), jnp.float32),
            pltpu.SemaphoreType.DMA,
        ],
    )
    def sc_gather(cm_hbm, b1m_hbm, b2m_hbm, lp_hbm, out_hbm,
                  seq_v, lp_v, out_v, sem):
        wid = lax.axis_index("s") * 2 + lax.axis_index("c")
        q = wid // 8
        g = wid % 8
        row0 = g * GS

        @pl.when(q == 0)
        def _():
            for k in range(GS):
                pltpu.make_async_copy(
                    cm_hbm.at[row0 + k, pl.ds(0, QL)],
                    seq_v.at[pl.ds(k * QL, QL)], sem).start()

        @pl.when(q == 1)
        def _():
            for k in range(GS):
                pltpu.make_async_copy(
                    cm_hbm.at[row0 + k, pl.ds(QL, QL)],
                    seq_v.at[pl.ds(k * QL, QL)], sem).start()

        @pl.when(q == 2)
        def _():
            for k in range(GS):
                pltpu.make_async_copy(
                    b1m_hbm.at[row0 + k, :],
                    seq_v.at[pl.ds(k * QL, QL)], sem).start()

        @pl.when(q == 3)
        def _():
            for k in range(GS):
                pltpu.make_async_copy(
                    b2m_hbm.at[row0 + k, :],
                    seq_v.at[pl.ds(k * QL, QL)], sem).start()

        pltpu.sync_copy(lp_hbm.at[pl.ds(q * QL * TS, QL * TS)], lp_v)
        for k in range(GS):
            pltpu.make_async_copy(
                cm_hbm.at[row0 + k, pl.ds(0, QL)],
                seq_v.at[pl.ds(k * QL, QL)], sem).wait()

        lane = lax.broadcasted_iota(jnp.int32, (16,), 0)
        pos_a = lane * QL
        pos_b = pos_a + 16 * QL

        def body(l, accs):
            acc_a, acc_b = accs
            sva = plsc.load_gather(seq_v, [pos_a + l])
            svb = plsc.load_gather(seq_v, [pos_b + l])
            tbase = l * TS
            ta = plsc.load_gather(lp_v, [sva + tbase])
            tb = plsc.load_gather(lp_v, [svb + tbase])
            return acc_a + ta, acc_b + tb

        zero = jnp.zeros((16,), jnp.float32)
        acc_a, acc_b = lax.fori_loop(0, QL, body, (zero, zero), unroll=8)
        out_v[pl.ds(0, 16)] = acc_a
        out_v[pl.ds(16, 16)] = acc_b
        pltpu.sync_copy(out_v, out_hbm.at[q, pl.ds(row0, GS)])

    return sc_gather


def kernel(complex_x, binder1_x, binder2_x, W,
           complex_mut_seqs, binder1_mut_seqs, binder2_mut_seqs,
           complex_wt_seq, binder1_wt_seq, binder2_wt_seq):
    wt_all = jnp.concatenate(
        [complex_wt_seq, binder1_wt_seq, binder2_wt_seq]
    ).astype(jnp.int32).reshape(LTOT, 1)

    lp_adj = pl.pallas_call(
        _tc_logprob_body,
        out_shape=jax.ShapeDtypeStruct((LTOT, V), jnp.float32),
    )(complex_x, binder1_x, binder2_x, W, wt_all)

    partials = _make_sc_gather()(
        complex_mut_seqs.astype(jnp.int32),
        binder1_mut_seqs.astype(jnp.int32),
        binder2_mut_seqs.astype(jnp.int32),
        lp_adj.reshape(LTOT * TS),
    )
    return (partials[0] + partials[1]) + (partials[2] + partials[3])
